# baseline (speedup 1.0000x reference)
import functools

import jax
import jax.numpy as jnp
from jax import lax
from jax.experimental import pallas as pl
from jax.experimental.pallas import tpu as pltpu
from jax.experimental.pallas import tpu_sc as plsc

NC, NS, L = 2, 16, 16
CS = 128
NCOLS = 16
NTS = 2
T = 128
NBUF = 4
LEAD = 2


def _make_sc_call(N, D, E):
    TH = N // NTS
    n_chunks = TH // T
    n_groups = n_chunks // NBUF

    mesh = plsc.VectorSubcoreMesh(core_axis_name="c", subcore_axis_name="s")

    @functools.partial(
        pl.kernel,
        out_type=jax.ShapeDtypeStruct((N, D), jnp.float32),
        mesh=mesh,
        scratch_types=[
            pltpu.VMEM((E, CS), jnp.float32),
            pltpu.VMEM((TH + L,), jnp.int32),
            pltpu.VMEM((TH + L,), jnp.float32),
            [pltpu.VMEM((T, CS), jnp.float32) for _ in range(NBUF)],
            pltpu.SemaphoreType.DMA((NBUF,)),
            pltpu.SemaphoreType.DMA((NBUF,)),
            pltpu.SemaphoreType.DMA,
            pltpu.VMEM_SHARED((32, 2048), jnp.float32),
            pltpu.SemaphoreType.DMA,
        ],
    )
    def call(hs_hbm, expert_hbm, routes_hbm, prob_hbm, out_hbm,
             table_v, routes_v, probs_v, bufs, in_sems, out_sems, st_sem,
             spbuf, sp_sem):
        wid = lax.axis_index("s") * NC + lax.axis_index("c")
        row0 = (wid // NCOLS) * TH
        col0 = (wid % NCOLS) * CS

        stage = [
            pltpu.async_copy(expert_hbm.at[:, pl.ds(col0, CS)], table_v,
                             st_sem),
            pltpu.async_copy(routes_hbm.at[pl.ds(row0, TH)],
                             routes_v.at[pl.ds(0, TH)], st_sem),
            pltpu.async_copy(prob_hbm.at[pl.ds(row0, TH)],
                             probs_v.at[pl.ds(0, TH)], st_sem),
        ]

        def hbm_block(c):
            return (pl.ds(row0 + c * T, T), pl.ds(col0, CS))

        def issue_in(c, b):
            r, cc = hbm_block(c)
            pltpu.async_copy(hs_hbm.at[r, cc], bufs[b], in_sems.at[b])

        def wait_in(c, b):
            r, cc = hbm_block(c)
            pltpu.make_async_copy(hs_hbm.at[r, cc], bufs[b],
                                  in_sems.at[b]).wait()

        def issue_out(c, b):
            r, cc = hbm_block(c)
            pltpu.async_copy(bufs[b], out_hbm.at[r, cc], out_sems.at[b])

        def wait_out(b):
            r, cc = hbm_block(0)
            pltpu.make_async_copy(bufs[b], out_hbm.at[r, cc],
                                  out_sems.at[b]).wait()

        def compute(c, b):
            hb = bufs[b]
            lbase = c * T

            @plsc.parallel_loop(0, T, 1, unroll=1)
            def tok_body(t):
                lt = lbase + t
                rv = routes_v[pl.ds(lt, L)]
                pvv = probs_v[pl.ds(lt, L)]
                rt = rv[0]
                pv = jnp.broadcast_to(pvv[0], (L,))
                for j in range(CS // L):
                    sl = pl.ds(j * L, L)
                    pass

        def chunk_body(c, b, prefetch, head):
            if prefetch:
                pb = (b + LEAD) % NBUF
                if not head:
                    wait_out(pb)
                issue_in(c + LEAD, pb)
            wait_in(c, b)
            compute(c, b)
            issue_out(c, b)

        @pl.when(wid < 2)
        def _probe():
            for k in range(64):
                pltpu.async_copy(hs_hbm.at[pl.ds(32 * k, 32), :], spbuf,
                                 sp_sem)

        for b in range(LEAD):
            issue_in(b, b)

        for d_ in stage:
            d_.wait()

        for b in range(NBUF):
            chunk_body(b, b, prefetch=True, head=(b < NBUF - LEAD))

        def group_body(g, carry):
            base = g * NBUF
            for b in range(NBUF):
                chunk_body(base + b, b, prefetch=True, head=False)
            return carry

        lax.fori_loop(1, n_groups - 1, group_body, 0)

        base = (n_groups - 1) * NBUF
        for b in range(NBUF):
            c = base + b
            chunk_body(c, b, prefetch=(b < NBUF - LEAD), head=False)

        for b in range(NBUF):
            wait_out(b)

        @pl.when(wid < 2)
        def _probe_drain():
            for k in range(64):
                pltpu.make_async_copy(hs_hbm.at[pl.ds(0, 32), :], spbuf,
                                      sp_sem).wait()

    return call


def kernel(hidden_states, expert_output, routes, route_prob_max):
    b, s, d = hidden_states.shape
    e = expert_output.shape[0]
    n = b * s
    hs2 = hidden_states.reshape(n, d)
    routes_i32 = routes.astype(jnp.int32)
    out = _make_sc_call(n, d, e)(
        hs2, expert_output, routes_i32, route_prob_max)
    return out.reshape(b, s, d)


# speedup vs baseline: 1.0055x; 1.0055x over previous
import functools

import jax
import jax.numpy as jnp
from jax import lax
from jax.experimental import pallas as pl
from jax.experimental.pallas import tpu as pltpu
from jax.experimental.pallas import tpu_sc as plsc

NC, NS, L = 2, 16, 16
CS = 128
NCOLS = 16
NTS = 2
T = 128
NBUF = 4
LEAD = 2


def _make_sc_call(N, D, E):
    TH = N // NTS
    n_chunks = TH // T
    n_groups = n_chunks // NBUF

    mesh = plsc.VectorSubcoreMesh(core_axis_name="c", subcore_axis_name="s")

    @functools.partial(
        pl.kernel,
        out_type=jax.ShapeDtypeStruct((N, D), jnp.float32),
        mesh=mesh,
        scratch_types=[
            pltpu.VMEM((E, CS), jnp.float32),
            pltpu.VMEM((TH + L,), jnp.int32),
            pltpu.VMEM((TH + L,), jnp.float32),
            [pltpu.VMEM((T, CS), jnp.float32) for _ in range(NBUF)],
            pltpu.SemaphoreType.DMA((NBUF,)),
            pltpu.SemaphoreType.DMA((NBUF,)),
            pltpu.SemaphoreType.DMA,
        ],
    )
    def call(hs_hbm, expert_hbm, routes_hbm, prob_hbm, out_hbm,
             table_v, routes_v, probs_v, bufs, in_sems, out_sems, st_sem):
        wid = lax.axis_index("s") * NC + lax.axis_index("c")
        row0 = (wid // NCOLS) * TH
        col0 = (wid % NCOLS) * CS

        stage = [
            pltpu.async_copy(expert_hbm.at[:, pl.ds(col0, CS)], table_v,
                             st_sem),
            pltpu.async_copy(routes_hbm.at[pl.ds(row0, TH)],
                             routes_v.at[pl.ds(0, TH)], st_sem),
            pltpu.async_copy(prob_hbm.at[pl.ds(row0, TH)],
                             probs_v.at[pl.ds(0, TH)], st_sem),
        ]

        def hbm_block(c):
            return (pl.ds(row0 + c * T, T), pl.ds(col0, CS))

        def issue_in(c, b):
            r, cc = hbm_block(c)
            pltpu.async_copy(hs_hbm.at[r, cc], bufs[b], in_sems.at[b])

        def wait_in(c, b):
            r, cc = hbm_block(c)
            pltpu.make_async_copy(hs_hbm.at[r, cc], bufs[b],
                                  in_sems.at[b]).wait()

        T2 = T // 2

        def issue_out_half(c, b, h):
            rows = pl.ds(row0 + c * T + h * T2, T2)
            pltpu.async_copy(bufs[b].at[pl.ds(h * T2, T2)],
                             out_hbm.at[rows, pl.ds(col0, CS)],
                             out_sems.at[b])

        def wait_out(b):
            for h in range(2):
                rows = pl.ds(row0 + h * T2, T2)
                pltpu.make_async_copy(bufs[b].at[pl.ds(h * T2, T2)],
                                      out_hbm.at[rows, pl.ds(col0, CS)],
                                      out_sems.at[b]).wait()

        def compute(c, b, h):
            hb = bufs[b]
            lbase = c * T
            T2_ = T // 2

            @plsc.parallel_loop(h * T2_, (h + 1) * T2_, 1, unroll=1)
            def tok_body(t):
                lt = lbase + t
                rv = routes_v[pl.ds(lt, L)]
                pvv = probs_v[pl.ds(lt, L)]
                rt = rv[0]
                pv = jnp.broadcast_to(pvv[0], (L,))
                for j in range(CS // L):
                    sl = pl.ds(j * L, L)
                    plsc.addupdate(hb.at[t, sl], table_v[rt, sl] * pv)

        def chunk_body(c, b, prefetch, head):
            if prefetch:
                pb = (b + LEAD) % NBUF
                if not head:
                    wait_out(pb)
                issue_in(c + LEAD, pb)
            wait_in(c, b)
            compute(c, b, 0)
            issue_out_half(c, b, 0)
            compute(c, b, 1)
            issue_out_half(c, b, 1)

        for b in range(LEAD):
            issue_in(b, b)

        for d_ in stage:
            d_.wait()

        for b in range(NBUF):
            chunk_body(b, b, prefetch=True, head=(b < NBUF - LEAD))

        def group_body(g, carry):
            base = g * NBUF
            for b in range(NBUF):
                chunk_body(base + b, b, prefetch=True, head=False)
            return carry

        lax.fori_loop(1, n_groups - 1, group_body, 0)

        base = (n_groups - 1) * NBUF
        for b in range(NBUF):
            c = base + b
            chunk_body(c, b, prefetch=(b < NBUF - LEAD), head=False)

        for b in range(NBUF):
            wait_out(b)

    return call


def kernel(hidden_states, expert_output, routes, route_prob_max):
    b, s, d = hidden_states.shape
    e = expert_output.shape[0]
    n = b * s
    hs2 = hidden_states.reshape(n, d)
    routes_i32 = routes.astype(jnp.int32)
    out = _make_sc_call(n, d, e)(
        hs2, expert_output, routes_i32, route_prob_max)
    return out.reshape(b, s, d)


# speedup vs baseline: 1.0206x; 1.0150x over previous
import functools

import jax
import jax.numpy as jnp
from jax import lax
from jax.experimental import pallas as pl
from jax.experimental.pallas import tpu as pltpu
from jax.experimental.pallas import tpu_sc as plsc

NC, NS, L = 2, 16, 16
CS = 128
NCOLS = 16
NTS = 2
T = 128
NBUF = 4
LEAD = 2


def _make_sc_call(N, D, E):
    TH = N // NTS
    n_chunks = TH // T
    n_groups = n_chunks // NBUF

    mesh = plsc.VectorSubcoreMesh(core_axis_name="c", subcore_axis_name="s")

    @functools.partial(
        pl.kernel,
        out_type=jax.ShapeDtypeStruct((N, D), jnp.float32),
        mesh=mesh,
        scratch_types=[
            pltpu.VMEM((E, CS), jnp.float32),
            pltpu.VMEM((TH + L,), jnp.int32),
            pltpu.VMEM((TH + L,), jnp.float32),
            [pltpu.VMEM((T, CS), jnp.float32) for _ in range(NBUF)],
            pltpu.SemaphoreType.DMA((NBUF,)),
            pltpu.SemaphoreType.DMA((NBUF,)),
            pltpu.SemaphoreType.DMA,
        ],
    )
    def call(hs_hbm, expert_hbm, routes_hbm, prob_hbm, out_hbm,
             table_v, routes_v, probs_v, bufs, in_sems, out_sems, st_sem):
        wid = lax.axis_index("s") * NC + lax.axis_index("c")
        row0 = (wid // NCOLS) * TH
        col0 = (wid % NCOLS) * CS

        stage = [
            pltpu.async_copy(expert_hbm.at[:, pl.ds(col0, CS)], table_v,
                             st_sem),
            pltpu.async_copy(routes_hbm.at[pl.ds(row0, TH)],
                             routes_v.at[pl.ds(0, TH)], st_sem),
            pltpu.async_copy(prob_hbm.at[pl.ds(row0, TH)],
                             probs_v.at[pl.ds(0, TH)], st_sem),
        ]

        def hbm_block(c):
            return (pl.ds(row0 + c * T, T), pl.ds(col0, CS))

        def issue_in(c, b):
            r, cc = hbm_block(c)
            pltpu.async_copy(hs_hbm.at[r, cc], bufs[b], in_sems.at[b])

        def wait_in(c, b):
            r, cc = hbm_block(c)
            pltpu.make_async_copy(hs_hbm.at[r, cc], bufs[b],
                                  in_sems.at[b]).wait()

        def issue_out(c, b):
            r, cc = hbm_block(c)
            pltpu.async_copy(bufs[b], out_hbm.at[r, cc], out_sems.at[b])

        def wait_out(b):
            r, cc = hbm_block(0)
            pltpu.make_async_copy(bufs[b], out_hbm.at[r, cc],
                                  out_sems.at[b]).wait()

        def compute(c, b):
            hb = bufs[b]
            lbase = c * T

            @plsc.parallel_loop(0, T, 2, unroll=1)
            def tok_body(t):
                lt = lbase + t
                rv = routes_v[pl.ds(lt, L)]
                pvv = probs_v[pl.ds(lt, L)]
                for u in range(2):
                    rt = rv[u]
                    pv = jnp.broadcast_to(pvv[u], (L,))
                    for j in range(CS // L):
                        sl = pl.ds(j * L, L)
                        plsc.addupdate(hb.at[t + u, sl],
                                       table_v[rt, sl] * pv)

        def chunk_body(c, b, prefetch, head):
            if prefetch:
                pb = (b + LEAD) % NBUF
                if not head:
                    wait_out(pb)
                issue_in(c + LEAD, pb)
            wait_in(c, b)
            compute(c, b)
            issue_out(c, b)

        for b in range(LEAD):
            issue_in(b, b)

        for d_ in stage:
            d_.wait()

        for b in range(NBUF):
            chunk_body(b, b, prefetch=True, head=(b < NBUF - LEAD))

        def group_body(g, carry):
            base = g * NBUF
            for b in range(NBUF):
                chunk_body(base + b, b, prefetch=True, head=False)
            return carry

        lax.fori_loop(1, n_groups - 1, group_body, 0)

        base = (n_groups - 1) * NBUF
        for b in range(NBUF):
            c = base + b
            chunk_body(c, b, prefetch=(b < NBUF - LEAD), head=False)

        for b in range(NBUF):
            wait_out(b)

    return call


def kernel(hidden_states, expert_output, routes, route_prob_max):
    b, s, d = hidden_states.shape
    e = expert_output.shape[0]
    n = b * s
    hs2 = hidden_states.reshape(n, d)
    routes_i32 = routes.astype(jnp.int32)
    out = _make_sc_call(n, d, e)(
        hs2, expert_output, routes_i32, route_prob_max)
    return out.reshape(b, s, d)


# speedup vs baseline: 1.0595x; 1.0381x over previous
import functools

import jax
import jax.numpy as jnp
from jax import lax
from jax.experimental import pallas as pl
from jax.experimental.pallas import tpu as pltpu
from jax.experimental.pallas import tpu_sc as plsc

NC, NS, L = 2, 16, 16
CS = 128
NCOLS = 16
NTS = 2
T = 128
NBUF = 4
LEAD = 2


def _make_sc_call(N, D, E):
    TH = N // NTS
    n_chunks = TH // T
    n_groups = n_chunks // NBUF

    mesh = plsc.VectorSubcoreMesh(core_axis_name="c", subcore_axis_name="s")

    @functools.partial(
        pl.kernel,
        out_type=jax.ShapeDtypeStruct((N, D), jnp.float32),
        mesh=mesh,
        scratch_types=[
            pltpu.VMEM((E, CS), jnp.float32),
            pltpu.VMEM((TH + L,), jnp.int32),
            pltpu.VMEM((TH + L,), jnp.float32),
            [pltpu.VMEM((T, CS), jnp.float32) for _ in range(NBUF)],
            pltpu.SemaphoreType.DMA((NBUF,)),
            pltpu.SemaphoreType.DMA((NBUF,)),
            pltpu.SemaphoreType.DMA,
        ],
    )
    def call(hs_hbm, expert_hbm, routes_hbm, prob_hbm, out_hbm,
             table_v, routes_v, probs_v, bufs, in_sems, out_sems, st_sem):
        wid = lax.axis_index("s") * NC + lax.axis_index("c")
        row0 = (wid // NCOLS) * TH
        col0 = (wid % NCOLS) * CS

        stage = [
            pltpu.async_copy(expert_hbm.at[:, pl.ds(col0, CS)], table_v,
                             st_sem),
            pltpu.async_copy(routes_hbm.at[pl.ds(row0, TH)],
                             routes_v.at[pl.ds(0, TH)], st_sem),
            pltpu.async_copy(prob_hbm.at[pl.ds(row0, TH)],
                             probs_v.at[pl.ds(0, TH)], st_sem),
        ]

        def hbm_block(c):
            return (pl.ds(row0 + c * T, T), pl.ds(col0, CS))

        def issue_in(c, b):
            r, cc = hbm_block(c)
            pltpu.async_copy(hs_hbm.at[r, cc], bufs[b], in_sems.at[b])

        def wait_in(c, b):
            r, cc = hbm_block(c)
            pltpu.make_async_copy(hs_hbm.at[r, cc], bufs[b],
                                  in_sems.at[b]).wait()

        def issue_out(c, b):
            r, cc = hbm_block(c)
            pltpu.async_copy(bufs[b], out_hbm.at[r, cc], out_sems.at[b])

        def wait_out(b):
            r, cc = hbm_block(0)
            pltpu.make_async_copy(bufs[b], out_hbm.at[r, cc],
                                  out_sems.at[b]).wait()

        def compute(c, b):
            hb = bufs[b]
            lbase = c * T

            @plsc.parallel_loop(0, T, 1, unroll=1)
            def tok_body(t):
                lt = lbase + t
                rv = routes_v[pl.ds(lt, L)]
                pvv = probs_v[pl.ds(lt, L)]
                rt = rv[0]
                pv = jnp.broadcast_to(pvv[0], (L,))
                for j in range(CS // L):
                    sl = pl.ds(j * L, L)
                    plsc.addupdate(hb.at[t, sl], table_v[rt, sl] * pv)

        def chunk_body(c, b, prefetch, head):
            if prefetch:
                pb = (b + LEAD) % NBUF
                if not head:
                    wait_out(pb)
                issue_in(c + LEAD, pb)
            wait_in(c, b)
            compute(c, b)
            issue_out(c, b)

        for b in range(LEAD):
            issue_in(b, b)

        for d_ in stage:
            d_.wait()

        for b in range(NBUF):
            chunk_body(b, b, prefetch=True, head=(b < NBUF - LEAD))

        def group_body(g, carry):
            base = g * NBUF
            for b in range(NBUF):
                chunk_body(base + b, b, prefetch=True, head=False)
            return carry

        lax.fori_loop(1, n_groups - 1, group_body, 0)

        base = (n_groups - 1) * NBUF
        for b in range(NBUF):
            c = base + b
            chunk_body(c, b, prefetch=(b < NBUF - LEAD), head=False)

        for b in range(NBUF):
            wait_out(b)

    return call


def kernel(hidden_states, expert_output, routes, route_prob_max):
    b, s, d = hidden_states.shape
    e = expert_output.shape[0]
    n = b * s
    hs2 = hidden_states.reshape(n, d)
    routes_i32 = routes.astype(jnp.int32)
    out = _make_sc_call(n, d, e)(
        hs2, expert_output, routes_i32, route_prob_max)
    return out.reshape(b, s, d)
